# baseline (speedup 1.0000x reference)
import functools
from math import sqrt

import jax
import jax.numpy as jnp
from jax import lax
from jax.experimental import pallas as pl
from jax.experimental.pallas import tpu as pltpu
from jax.experimental.pallas import tpu_sc as plsc

_K = 16
_EPS = 1e-5
_SCALE = sqrt(128.0)


def _sc_gather(table, idx):
    T, D = table.shape
    (Btot,) = idx.shape
    info = plsc.get_sparse_core_info()
    nw = info.num_cores * info.num_subcores
    bpw = Btot // nw
    ch = min(bpw, 256)
    nch = bpw // ch
    mesh = plsc.VectorSubcoreMesh(core_axis_name="c", subcore_axis_name="s")

    @functools.partial(
        pl.kernel,
        mesh=mesh,
        out_type=jax.ShapeDtypeStruct((Btot, D), jnp.float32),
        scratch_types=[
            pltpu.VMEM((ch,), jnp.int32),
            pltpu.VMEM((ch, D), jnp.float32),
            pltpu.SemaphoreType.DMA,
        ],
    )
    def k(table_hbm, idx_hbm, out_hbm, idx_v, rows_v, sem):
        wid = lax.axis_index("s") * info.num_cores + lax.axis_index("c")
        base = wid * bpw

        def body(t, carry):
            off = pl.multiple_of(base + t * ch, 8)
            pltpu.sync_copy(idx_hbm.at[pl.ds(off, ch)], idx_v)
            pltpu.async_copy(table_hbm.at[idx_v], rows_v, sem).wait()
            pltpu.sync_copy(rows_v, out_hbm.at[pl.ds(off, ch)])
            return carry

        lax.fori_loop(0, nch, body, 0)

    return k(table, idx)


def _qkv_call(feats, xsmall, w1, w2, bias, widths):
    BN, D1 = feats.shape
    D2 = xsmall.shape[1]
    DOUT = w1.shape[1]
    R = 256
    offs = [sum(widths[:i]) for i in range(len(widths))]

    def body(x_ref, s_ref, w1_ref, w2_ref, b_ref, *out_refs):
        y = (jnp.dot(x_ref[...], w1_ref[...],
                     preferred_element_type=jnp.float32)
             + jnp.dot(s_ref[...], w2_ref[...],
                       preferred_element_type=jnp.float32)
             + b_ref[...])
        for o_ref, off, wd in zip(out_refs, offs, widths):
            o_ref[...] = y[:, off:off + wd]
        s = s_ref[...]
        out_refs[-1][:, 0:8] = jnp.concatenate(
            [s[:, 2:6], jnp.zeros((R, 4), jnp.float32)], axis=1)

    return pl.pallas_call(
        body,
        grid=(BN // R,),
        in_specs=[
            pl.BlockSpec((R, D1), lambda i: (i, 0)),
            pl.BlockSpec((R, D2), lambda i: (i, 0)),
            pl.BlockSpec((D1, DOUT), lambda i: (0, 0)),
            pl.BlockSpec((D2, DOUT), lambda i: (0, 0)),
            pl.BlockSpec((1, DOUT), lambda i: (0, 0)),
        ],
        out_specs=[pl.BlockSpec((R, wd), lambda i: (i, 0)) for wd in widths],
        out_shape=[jax.ShapeDtypeStruct((BN, wd), jnp.float32)
                   for wd in widths],
    )(feats, xsmall, w1, w2, bias)


def _local_ball_idx_call(xc, yc, zc, ptsT, r2, npts, bsel):
    R = 256
    nb = npts // R

    def body(x_ref, y_ref, z_ref, t_ref, li_ref, bi_ref):
        b = bsel
        j = pl.program_id(0)
        t = t_ref[0]
        dx = x_ref[...] - t[0:1, :]
        d2 = dx * dx
        dy = y_ref[...] - t[1:2, :]
        d2 = d2 + dy * dy
        dz = z_ref[...] - t[2:3, :]
        d3 = d2 + dz * dz
        iotaf = lax.broadcasted_iota(jnp.int32, (R, npts), 1).astype(
            jnp.float32)
        base = b * npts
        d = d3
        cols = []
        for _ in range(_K):
            m = jnp.min(d, axis=1, keepdims=True)
            jj = jnp.min(jnp.where(d == m, iotaf, float(npts)),
                         axis=1, keepdims=True)
            cols.append(jj.astype(jnp.int32) + base)
            d = jnp.where(iotaf == jj, jnp.inf, d)
        li_ref[...] = jnp.concatenate(cols, axis=1)
        key = jnp.where(d2 < r2, iotaf, iotaf + npts)
        rowi = (base + j * R
                + lax.broadcasted_iota(jnp.int32, (R, 1), 0))
        cols = []
        for _ in range(_K):
            m = jnp.min(key, axis=1, keepdims=True)
            mi = m.astype(jnp.int32)
            cols.append(jnp.where(mi < npts, mi + base, rowi))
            key = jnp.where(key == m, float(2 * npts), key)
        bi_ref[...] = jnp.concatenate(cols, axis=1)

    out_sh = jax.ShapeDtypeStruct((npts, _K), jnp.int32)
    return pl.pallas_call(
        body,
        grid=(nb,),
        in_specs=[
            pl.BlockSpec((R, 1), lambda j: (bsel * nb + j, 0)),
            pl.BlockSpec((R, 1), lambda j: (bsel * nb + j, 0)),
            pl.BlockSpec((R, 1), lambda j: (bsel * nb + j, 0)),
            pl.BlockSpec((1, 8, npts), lambda j: (bsel, 0, 0)),
        ],
        out_specs=[
            pl.BlockSpec((R, _K), lambda j: (j, 0)),
            pl.BlockSpec((R, _K), lambda j: (j, 0)),
        ],
        out_shape=[out_sh, out_sh],
    )(xc, yc, zc, ptsT)


def _knn_call(cxc, cyc, czc, ptsT, R, base_mult):
    nbatch = ptsT.shape[0]
    np_ = ptsT.shape[2]
    nc = cxc.shape[0] // nbatch
    nb = np_ // R

    def body(x_ref, y_ref, z_ref, t_ref, o_ref):
        b = pl.program_id(0)
        t = t_ref[0]
        dx = x_ref[...] - t[0:1, :]
        d = dx * dx
        dy = y_ref[...] - t[1:2, :]
        d = d + dy * dy
        dz = z_ref[...] - t[2:3, :]
        d = d + dz * dz
        iotaf = lax.broadcasted_iota(jnp.int32, (nc, R), 0).astype(
            jnp.float32)
        base = b * base_mult
        cols = []
        for _ in range(_K):
            m = jnp.min(d, axis=0, keepdims=True)
            jj = jnp.min(jnp.where(d == m, iotaf, float(nc)),
                         axis=0, keepdims=True)
            cols.append(jj.astype(jnp.int32) + base)
            d = jnp.where(iotaf == jj, jnp.inf, d)
        o_ref[...] = jnp.concatenate(cols, axis=0)

    return pl.pallas_call(
        body,
        grid=(nbatch, nb),
        in_specs=[
            pl.BlockSpec((nc, 1), lambda b, j: (b, 0)),
            pl.BlockSpec((nc, 1), lambda b, j: (b, 0)),
            pl.BlockSpec((nc, 1), lambda b, j: (b, 0)),
            pl.BlockSpec((1, 8, R), lambda b, j: (b, 0, j)),
        ],
        out_specs=pl.BlockSpec((_K, R), lambda b, j: (0, b * nb + j)),
        out_shape=jax.ShapeDtypeStruct((_K, nbatch * np_), jnp.int32),
    )(cxc, cyc, czc, ptsT)


def _fps_call(xsb, nbatch, nsamp):
    _, npts = xsb.shape

    def body(t_ref, o_ref):
        x0 = t_ref[0:nbatch, :]
        x1 = t_ref[nbatch:2 * nbatch, :]
        x2 = t_ref[2 * nbatch:3 * nbatch, :]
        iota = lax.broadcasted_iota(jnp.int32, (nbatch, npts), 1)
        iotaf = iota.astype(jnp.float32)
        for b in range(nbatch):
            o_ref[0:1, b:b + 1] = jnp.full((1, 1), b * npts, jnp.int32)

        iota12 = lax.broadcasted_iota(jnp.int32, (3 * nbatch, npts), 1)
        x12 = t_ref[0:3 * nbatch, :]

        def step(i, carry):
            dists, last = carry
            mask = iota == last
            last12 = jnp.concatenate([last, last, last], axis=0)
            l12 = jnp.sum(
                jnp.where(iota12 == last12, x12, 0.0),
                axis=1, keepdims=True)
            l0 = l12[0:nbatch]
            l1 = l12[nbatch:2 * nbatch]
            l2 = l12[2 * nbatch:3 * nbatch]
            s0 = (x0 - l0) * (x0 - l0)
            s1 = (x1 - l1) * (x1 - l1)
            s2 = (x2 - l2) * (x2 - l2)
            d = (s0 + s1) + s2
            dists = jnp.minimum(dists, d)
            mx = jnp.max(dists, axis=1, keepdims=True)
            cand = jnp.where(dists == mx, iotaf, float(npts))
            nxt = jnp.min(cand, axis=1, keepdims=True).astype(jnp.int32)
            for b in range(nbatch):
                o_ref[pl.ds(i, 1), b:b + 1] = nxt[b:b + 1, :] + b * npts
            return dists, nxt

        lax.fori_loop(
            1, nsamp, step,
            (jnp.full((nbatch, npts), jnp.inf, jnp.float32),
             jnp.zeros((nbatch, 1), jnp.int32)),
        )

    return pl.pallas_call(
        body,
        grid=(1,),
        in_specs=[pl.BlockSpec((16, npts), lambda b: (0, 0))],
        out_specs=pl.BlockSpec((nsamp, 8), lambda b: (0, 0)),
        out_shape=jax.ShapeDtypeStruct((nsamp, 8), jnp.int32),
    )(xsb)


def _maxpool_call(x, d, extra):
    Btot = x.shape[0] // _K
    e = extra.shape[1]
    R = 64

    def body(x_ref, e_ref, o_ref):
        o_ref[:, 0:d] = jnp.max(x_ref[...].reshape(R, _K, d), axis=1)
        o_ref[:, d:d + e] = e_ref[...]

    return pl.pallas_call(
        body,
        grid=(Btot // R,),
        in_specs=[pl.BlockSpec((R * _K, d), lambda i: (i, 0)),
                  pl.BlockSpec((R, e), lambda i: (i, 0))],
        out_specs=pl.BlockSpec((R, d + e), lambda i: (i, 0)),
        out_shape=jax.ShapeDtypeStruct((Btot, d + e), jnp.float32),
    )(x, extra)


def _attn_call(qall, qi, call, ci, gath, b1, w2, b2, gnorm, bnorm, rb0=0):
    BN = qall.shape[0]
    R = 256

    def body(q_ref, own_ref, g_ref, b1_ref, w2_ref, b2_ref,
             gn_ref, bn_ref, o_ref):
        gg = g_ref[...]
        hpre = (own_ref[...].reshape(R, 1, 128)
                - gg[:, 256:384].reshape(R, _K, 128)
                + b1_ref[...].reshape(1, 1, 128))
        h = jnp.maximum(hpre, 0.0)
        pe = (jnp.dot(h.reshape(R * _K, 128), w2_ref[...],
                      preferred_element_type=jnp.float32)
              + b2_ref[...]).reshape(R, _K, 128)
        t = q_ref[...].reshape(R, 1, 128) - gg[:, 0:128].reshape(R, _K, 128) + pe
        onescol = jnp.full((128, 8), 1.0 / 128.0, jnp.float32)
        mu = jnp.dot(t.reshape(R * _K, 128), onescol,
                     preferred_element_type=jnp.float32)[:, 0:1] \
            .reshape(R, _K, 1)
        xcen = t - mu
        var = jnp.dot((xcen * xcen).reshape(R * _K, 128), onescol,
                      preferred_element_type=jnp.float32)[:, 0:1] \
            .reshape(R, _K, 1)
        a = (xcen / jnp.sqrt(var + _EPS)) * gn_ref[...].reshape(1, 1, 128) \
            + bn_ref[...].reshape(1, 1, 128)
        a = a / _SCALE
        mx = jnp.max(a, axis=1, keepdims=True)
        e = jnp.exp(a - mx)
        a = e / jnp.sum(e, axis=1, keepdims=True)
        v = gg[:, 128:256].reshape(R, _K, 128)
        o_ref[...] = jnp.sum(a * (v + pe), axis=1)

    const = lambda i: (0, 0)
    nrows = gath.shape[0] // _K
    return pl.pallas_call(
        body,
        grid=(nrows // R,),
        in_specs=[
            pl.BlockSpec((R, 128), lambda i: (rb0 + i, qi)),
            pl.BlockSpec((R, 128), lambda i: (rb0 + i, ci)),
            pl.BlockSpec((R * _K, 384), lambda i: (i, 0)),
            pl.BlockSpec((1, 128), const),
            pl.BlockSpec((128, 128), const),
            pl.BlockSpec((1, 128), const),
            pl.BlockSpec((1, 128), const),
            pl.BlockSpec((1, 128), const),
        ],
        out_specs=pl.BlockSpec((R, 128), lambda i: (i, 0)),
        out_shape=jax.ShapeDtypeStruct((nrows, 128), jnp.float32),
    )(qall, call, gath, b1, w2, b2, gnorm, bnorm)


def _proj_call(xs, w1, b1, w2, b2):
    BN = xs[0].shape[0]
    R = 256

    def body(x0_ref, x1_ref, x2_ref, w1_ref, b1_ref, w2_ref, b2_ref, o_ref):
        h = (jnp.dot(x0_ref[...], w1_ref[0:128],
                     preferred_element_type=jnp.float32)
             + jnp.dot(x1_ref[...], w1_ref[128:256],
                       preferred_element_type=jnp.float32)
             + jnp.dot(x2_ref[...], w1_ref[256:384],
                       preferred_element_type=jnp.float32)
             + b1_ref[...])
        h = jnp.maximum(h, 0.0)
        o_ref[...] = (jnp.dot(h, w2_ref[...],
                              preferred_element_type=jnp.float32) + b2_ref[...])

    const = lambda i: (0, 0)
    return pl.pallas_call(
        body,
        grid=(BN // R,),
        in_specs=[
            pl.BlockSpec((R, 128), lambda i: (i, 0)),
            pl.BlockSpec((R, 128), lambda i: (i, 0)),
            pl.BlockSpec((R, 128), lambda i: (i, 0)),
            pl.BlockSpec((384, 128), const),
            pl.BlockSpec((1, 128), const),
            pl.BlockSpec((128, 128), const),
            pl.BlockSpec((1, 128), const),
        ],
        out_specs=pl.BlockSpec((R, 128), lambda i: (i, 0)),
        out_shape=jax.ShapeDtypeStruct((BN, 128), jnp.float32),
    )(*xs, w1, b1, w2, b2)


def kernel(xyzp, features, params):
    p = params
    nbatch, npts, _ = xyzp.shape
    BN = nbatch * npts
    nsamp = npts // 8
    BM = nbatch * nsamp
    f32 = jnp.float32

    xy = xyzp[..., :2]
    ptsT = jnp.concatenate(
        [jnp.swapaxes(xyzp, 1, 2), jnp.zeros((nbatch, 4, npts), f32)], axis=1)
    xcol = xyzp[..., 0].reshape(BN, 1)
    ycol = xyzp[..., 1].reshape(BN, 1)
    zcol = xyzp[..., 2].reshape(BN, 1)

    lw, lb = p['local_qkv_w'], p['local_qkv_b']
    cwf, cwp, cb = p['conv_qkv_wf'], p['conv_qkv_wp'], p['conv_qkv_b']
    gw, gb = p['global_qkv_w'], p['global_qkv_b']
    zf = jnp.zeros((128, 128), f32)
    zxy = jnp.zeros((2, 128), f32)
    zp = jnp.zeros((4, 128), f32)
    zb = jnp.zeros((128,), f32)
    band_f = jnp.concatenate([
        lw[:, 128:256], lw[:, 256:384], zf,
        cwf[:, 128:256], cwf[:, 256:384], zf,
        gw[:, 128:256], gw[:, 256:384],
        lw[:, 0:128], cwf[:, 0:128], gw[:, 0:128],
        zf, zf, zf,
        jnp.zeros((128, 256), f32)], axis=1)
    band_s = jnp.concatenate([
        jnp.concatenate([zxy, zxy, zxy,
                         cwp[:, 128:256], cwp[:, 256:384], p['conv_pe_w1'],
                         zxy, zxy,
                         zxy, cwp[:, 0:128], zxy,
                         zxy, p['conv_pe_w1'], zxy,
                         jnp.zeros((2, 256), f32)], axis=1),
        jnp.concatenate([zp, zp, p['local_pe_w1'],
                         zp, zp, zp,
                         zp, zp,
                         zp, zp, zp,
                         p['local_pe_w1'], zp, p['global_pe_w1'],
                         jnp.concatenate([jnp.zeros((4, 128), f32),
                                          p['global_pe_w1']], axis=1)],
                        axis=1),
        jnp.zeros((2, 2048), f32)], axis=0)
    bias = jnp.concatenate([
        lb[128:256], lb[256:384], zb,
        cb[128:256], cb[256:384], zb,
        gb[128:256], gb[256:384],
        lb[0:128], cb[0:128], gb[0:128],
        zb, zb, zb, jnp.zeros((256,), f32)])[None, :]
    xsmall = jnp.concatenate([xy.reshape(BN, 2), xyzp.reshape(BN, 4),
                              jnp.zeros((BN, 2), f32)], axis=1)
    table_l, table_c, table_g, qall, c1all, down_table = _qkv_call(
        features.reshape(BN, 128), xsmall, band_f, band_s, bias,
        (384, 384, 256, 384, 384, 256))

    r2 = (5.0 / 128.0) ** 2
    lb_idx = [_local_ball_idx_call(xcol, ycol, zcol, ptsT, r2, npts, b)
              for b in range(nbatch)]

    xsb = jnp.transpose(xyzp, (2, 0, 1)).reshape(16, npts)
    down_flat = jnp.transpose(_fps_call(xsb, nbatch, nsamp)[:, :nbatch],
                              (1, 0)).reshape(BM)
    dg = _sc_gather(down_table, down_flat)
    down16 = dg[:, :16]
    down_c1g = dg[:, 128:256]
    d16 = down16.reshape(nbatch, nsamp, 16)
    downT = jnp.swapaxes(d16, 1, 2)[:, :8, :]
    dxc = d16[..., 0].reshape(BM, 1)
    dyc = d16[..., 1].reshape(BM, 1)
    dzc = d16[..., 2].reshape(BM, 1)
    pair = jnp.transpose(
        _knn_call(xcol, ycol, zcol, downT, min(nsamp, 256), npts),
        (1, 0))
    inv = jnp.transpose(
        _knn_call(dxc, dyc, dzc, ptsT, 512, nsamp), (1, 0))

    gd = _sc_gather(table_g, pair.reshape(BM * _K))
    table_g2 = _maxpool_call(gd, 256, down_c1g)

    def attn(name, bidx, g, rb0):
        return _attn_call(
            qall, bidx, c1all, bidx, g,
            p[name + '_pe_b1'][None],
            p[name + '_pe_w2'], p[name + '_pe_b2'][None],
            p[name + '_fc_g'][None], p[name + '_fc_b'][None], rb0=rb0)

    nbb = npts // 256
    ge = _sc_gather(table_g2, inv.reshape(BN * _K))
    parts = {'local': [], 'conv': []}
    for b in range(nbatch):
        li_b, bi_b = lb_idx[b]
        ga_b = _sc_gather(table_l, li_b.reshape(npts * _K))
        gb_b = _sc_gather(table_c, bi_b.reshape(npts * _K))
        parts['local'].append(attn('local', 0, ga_b, b * nbb))
        parts['conv'].append(attn('conv', 1, gb_b, b * nbb))
    glob = attn('global', 2, ge, 0)
    local = jnp.concatenate(parts['local'], axis=0)
    conv = jnp.concatenate(parts['conv'], axis=0)

    out = _proj_call([local, conv, glob],
                     p['proj_w1'], p['proj_b1'][None],
                     p['proj_w2'], p['proj_b2'][None])
    return out.reshape(nbatch, npts, 128)


# speedup vs baseline: 1.0134x; 1.0134x over previous
import functools
from math import sqrt

import jax
import jax.numpy as jnp
from jax import lax
from jax.experimental import pallas as pl
from jax.experimental.pallas import tpu as pltpu
from jax.experimental.pallas import tpu_sc as plsc

_K = 16
_EPS = 1e-5
_SCALE = sqrt(128.0)


def _sc_gather(table, idx):
    T, D = table.shape
    (Btot,) = idx.shape
    info = plsc.get_sparse_core_info()
    nw = info.num_cores * info.num_subcores
    bpw = Btot // nw
    ch = min(bpw, 256)
    nch = bpw // ch
    mesh = plsc.VectorSubcoreMesh(core_axis_name="c", subcore_axis_name="s")

    @functools.partial(
        pl.kernel,
        mesh=mesh,
        out_type=jax.ShapeDtypeStruct((Btot, D), jnp.float32),
        scratch_types=[
            pltpu.VMEM((ch,), jnp.int32),
            pltpu.VMEM((ch, D), jnp.float32),
            pltpu.SemaphoreType.DMA,
        ],
    )
    def k(table_hbm, idx_hbm, out_hbm, idx_v, rows_v, sem):
        wid = lax.axis_index("s") * info.num_cores + lax.axis_index("c")
        base = wid * bpw

        def body(t, carry):
            off = pl.multiple_of(base + t * ch, 8)
            pltpu.sync_copy(idx_hbm.at[pl.ds(off, ch)], idx_v)
            pltpu.async_copy(table_hbm.at[idx_v], rows_v, sem).wait()
            pltpu.sync_copy(rows_v, out_hbm.at[pl.ds(off, ch)])
            return carry

        lax.fori_loop(0, nch, body, 0)

    return k(table, idx)


def _qkv_call(feats, xsmall, w1, w2, bias, widths):
    BN, D1 = feats.shape
    D2 = xsmall.shape[1]
    DOUT = w1.shape[1]
    R = 256
    offs = [sum(widths[:i]) for i in range(len(widths))]

    def body(x_ref, s_ref, w1_ref, w2_ref, b_ref, *out_refs):
        y = (jnp.dot(x_ref[...], w1_ref[...],
                     preferred_element_type=jnp.float32)
             + jnp.dot(s_ref[...], w2_ref[...],
                       preferred_element_type=jnp.float32)
             + b_ref[...])
        for o_ref, off, wd in zip(out_refs, offs, widths):
            o_ref[...] = y[:, off:off + wd]
        s = s_ref[...]
        out_refs[-1][:, 0:8] = jnp.concatenate(
            [s[:, 2:6], jnp.zeros((R, 4), jnp.float32)], axis=1)

    return pl.pallas_call(
        body,
        grid=(BN // R,),
        in_specs=[
            pl.BlockSpec((R, D1), lambda i: (i, 0)),
            pl.BlockSpec((R, D2), lambda i: (i, 0)),
            pl.BlockSpec((D1, DOUT), lambda i: (0, 0)),
            pl.BlockSpec((D2, DOUT), lambda i: (0, 0)),
            pl.BlockSpec((1, DOUT), lambda i: (0, 0)),
        ],
        out_specs=[pl.BlockSpec((R, wd), lambda i: (i, 0)) for wd in widths],
        out_shape=[jax.ShapeDtypeStruct((BN, wd), jnp.float32)
                   for wd in widths],
    )(feats, xsmall, w1, w2, bias)


def _local_ball_idx_call(xc, yc, zc, ptsT, r2, npts, bsel):
    R = 256
    nb = npts // R

    def body(x_ref, y_ref, z_ref, t_ref, li_ref, bi_ref):
        b = bsel
        j = pl.program_id(0)
        t = t_ref[0]
        dx = x_ref[...] - t[0:1, :]
        d2 = dx * dx
        dy = y_ref[...] - t[1:2, :]
        d2 = d2 + dy * dy
        dz = z_ref[...] - t[2:3, :]
        d3 = d2 + dz * dz
        iotaf = lax.broadcasted_iota(jnp.int32, (R, npts), 1).astype(
            jnp.float32)
        base = b * npts
        d = d3
        cols = []
        for _ in range(_K):
            m = jnp.min(d, axis=1, keepdims=True)
            jj = jnp.min(jnp.where(d == m, iotaf, float(npts)),
                         axis=1, keepdims=True)
            cols.append(jj.astype(jnp.int32) + base)
            d = jnp.where(iotaf == jj, jnp.inf, d)
        li_ref[...] = jnp.concatenate(cols, axis=1)
        key = jnp.where(d2 < r2, iotaf, iotaf + npts)
        rowi = (base + j * R
                + lax.broadcasted_iota(jnp.int32, (R, 1), 0))
        cols = []
        for _ in range(_K):
            m = jnp.min(key, axis=1, keepdims=True)
            mi = m.astype(jnp.int32)
            cols.append(jnp.where(mi < npts, mi + base, rowi))
            key = jnp.where(key == m, float(2 * npts), key)
        bi_ref[...] = jnp.concatenate(cols, axis=1)

    out_sh = jax.ShapeDtypeStruct((npts, _K), jnp.int32)
    return pl.pallas_call(
        body,
        grid=(nb,),
        in_specs=[
            pl.BlockSpec((R, 1), lambda j: (bsel * nb + j, 0)),
            pl.BlockSpec((R, 1), lambda j: (bsel * nb + j, 0)),
            pl.BlockSpec((R, 1), lambda j: (bsel * nb + j, 0)),
            pl.BlockSpec((1, 8, npts), lambda j: (bsel, 0, 0)),
        ],
        out_specs=[
            pl.BlockSpec((R, _K), lambda j: (j, 0)),
            pl.BlockSpec((R, _K), lambda j: (j, 0)),
        ],
        out_shape=[out_sh, out_sh],
    )(xc, yc, zc, ptsT)


def _knn_call(cxc, cyc, czc, ptsT, R, base_mult):
    nbatch = ptsT.shape[0]
    np_ = ptsT.shape[2]
    nc = cxc.shape[0] // nbatch
    nb = np_ // R

    def body(x_ref, y_ref, z_ref, t_ref, o_ref):
        b = pl.program_id(0)
        t = t_ref[0]
        dx = x_ref[...] - t[0:1, :]
        d = dx * dx
        dy = y_ref[...] - t[1:2, :]
        d = d + dy * dy
        dz = z_ref[...] - t[2:3, :]
        d = d + dz * dz
        iotaf = lax.broadcasted_iota(jnp.int32, (nc, R), 0).astype(
            jnp.float32)
        base = b * base_mult
        cols = []
        for _ in range(_K):
            m = jnp.min(d, axis=0, keepdims=True)
            jj = jnp.min(jnp.where(d == m, iotaf, float(nc)),
                         axis=0, keepdims=True)
            cols.append(jj.astype(jnp.int32) + base)
            d = jnp.where(iotaf == jj, jnp.inf, d)
        o_ref[...] = jnp.concatenate(cols, axis=0)

    return pl.pallas_call(
        body,
        grid=(nbatch, nb),
        in_specs=[
            pl.BlockSpec((nc, 1), lambda b, j: (b, 0)),
            pl.BlockSpec((nc, 1), lambda b, j: (b, 0)),
            pl.BlockSpec((nc, 1), lambda b, j: (b, 0)),
            pl.BlockSpec((1, 8, R), lambda b, j: (b, 0, j)),
        ],
        out_specs=pl.BlockSpec((_K, R), lambda b, j: (0, b * nb + j)),
        out_shape=jax.ShapeDtypeStruct((_K, nbatch * np_), jnp.int32),
    )(cxc, cyc, czc, ptsT)


def _fps_call(xsb, nbatch, nsamp):
    _, npts = xsb.shape

    def body(t_ref, o_ref):
        x0 = t_ref[0:nbatch, :]
        x1 = t_ref[nbatch:2 * nbatch, :]
        x2 = t_ref[2 * nbatch:3 * nbatch, :]
        iota = lax.broadcasted_iota(jnp.int32, (nbatch, npts), 1)
        iotaf = iota.astype(jnp.float32)
        for b in range(nbatch):
            o_ref[0:1, b:b + 1] = jnp.full((1, 1), b * npts, jnp.int32)

        def step(i, carry):
            dists, last = carry
            mask = iota == last
            zero = jnp.zeros((nbatch, npts), jnp.float32)
            l0 = jnp.sum(jnp.where(mask, x0, zero), axis=1, keepdims=True)
            l1 = jnp.sum(jnp.where(mask, x1, zero), axis=1, keepdims=True)
            l2 = jnp.sum(jnp.where(mask, x2, zero), axis=1, keepdims=True)
            s0 = (x0 - l0) * (x0 - l0)
            s1 = (x1 - l1) * (x1 - l1)
            s2 = (x2 - l2) * (x2 - l2)
            d = (s0 + s1) + s2
            dists = jnp.minimum(dists, d)
            mx = jnp.max(dists, axis=1, keepdims=True)
            cand = jnp.where(dists == mx, iotaf, float(npts))
            nxt = jnp.min(cand, axis=1, keepdims=True).astype(jnp.int32)
            for b in range(nbatch):
                o_ref[pl.ds(i, 1), b:b + 1] = nxt[b:b + 1, :] + b * npts
            return dists, nxt

        lax.fori_loop(
            1, nsamp, step,
            (jnp.full((nbatch, npts), jnp.inf, jnp.float32),
             jnp.zeros((nbatch, 1), jnp.int32)),
        )

    return pl.pallas_call(
        body,
        grid=(1,),
        in_specs=[pl.BlockSpec((16, npts), lambda b: (0, 0))],
        out_specs=pl.BlockSpec((nsamp, 8), lambda b: (0, 0)),
        out_shape=jax.ShapeDtypeStruct((nsamp, 8), jnp.int32),
    )(xsb)


def _maxpool_call(x, d, extra):
    Btot = x.shape[0] // _K
    e = extra.shape[1]
    R = 64

    def body(x_ref, e_ref, o_ref):
        o_ref[:, 0:d] = jnp.max(x_ref[...].reshape(R, _K, d), axis=1)
        o_ref[:, d:d + e] = e_ref[...]

    return pl.pallas_call(
        body,
        grid=(Btot // R,),
        in_specs=[pl.BlockSpec((R * _K, d), lambda i: (i, 0)),
                  pl.BlockSpec((R, e), lambda i: (i, 0))],
        out_specs=pl.BlockSpec((R, d + e), lambda i: (i, 0)),
        out_shape=jax.ShapeDtypeStruct((Btot, d + e), jnp.float32),
    )(x, extra)


def _attn_call(qall, qi, call, ci, gath, b1, w2, b2, gnorm, bnorm, rb0=0):
    BN = qall.shape[0]
    R = 256

    def body(q_ref, own_ref, g_ref, b1_ref, w2_ref, b2_ref,
             gn_ref, bn_ref, o_ref):
        gg = g_ref[...]
        hpre = (own_ref[...].reshape(R, 1, 128)
                - gg[:, 256:384].reshape(R, _K, 128)
                + b1_ref[...].reshape(1, 1, 128))
        h = jnp.maximum(hpre, 0.0)
        pe = (jnp.dot(h.reshape(R * _K, 128), w2_ref[...],
                      preferred_element_type=jnp.float32)
              + b2_ref[...]).reshape(R, _K, 128)
        t = q_ref[...].reshape(R, 1, 128) - gg[:, 0:128].reshape(R, _K, 128) + pe
        onescol = jnp.full((128, 8), 1.0 / 128.0, jnp.float32)
        mu = jnp.dot(t.reshape(R * _K, 128), onescol,
                     preferred_element_type=jnp.float32)[:, 0:1] \
            .reshape(R, _K, 1)
        xcen = t - mu
        var = jnp.dot((xcen * xcen).reshape(R * _K, 128), onescol,
                      preferred_element_type=jnp.float32)[:, 0:1] \
            .reshape(R, _K, 1)
        a = (xcen / jnp.sqrt(var + _EPS)) * gn_ref[...].reshape(1, 1, 128) \
            + bn_ref[...].reshape(1, 1, 128)
        a = a / _SCALE
        mx = jnp.max(a, axis=1, keepdims=True)
        e = jnp.exp(a - mx)
        a = e / jnp.sum(e, axis=1, keepdims=True)
        v = gg[:, 128:256].reshape(R, _K, 128)
        o_ref[...] = jnp.sum(a * (v + pe), axis=1)

    const = lambda i: (0, 0)
    nrows = gath.shape[0] // _K
    return pl.pallas_call(
        body,
        grid=(nrows // R,),
        in_specs=[
            pl.BlockSpec((R, 128), lambda i: (rb0 + i, qi)),
            pl.BlockSpec((R, 128), lambda i: (rb0 + i, ci)),
            pl.BlockSpec((R * _K, 384), lambda i: (i, 0)),
            pl.BlockSpec((1, 128), const),
            pl.BlockSpec((128, 128), const),
            pl.BlockSpec((1, 128), const),
            pl.BlockSpec((1, 128), const),
            pl.BlockSpec((1, 128), const),
        ],
        out_specs=pl.BlockSpec((R, 128), lambda i: (i, 0)),
        out_shape=jax.ShapeDtypeStruct((nrows, 128), jnp.float32),
    )(qall, call, gath, b1, w2, b2, gnorm, bnorm)


def _proj_call(xs, w1, b1, w2, b2):
    BN = xs[0].shape[0]
    R = 256

    def body(x0_ref, x1_ref, x2_ref, w1_ref, b1_ref, w2_ref, b2_ref, o_ref):
        h = (jnp.dot(x0_ref[...], w1_ref[0:128],
                     preferred_element_type=jnp.float32)
             + jnp.dot(x1_ref[...], w1_ref[128:256],
                       preferred_element_type=jnp.float32)
             + jnp.dot(x2_ref[...], w1_ref[256:384],
                       preferred_element_type=jnp.float32)
             + b1_ref[...])
        h = jnp.maximum(h, 0.0)
        o_ref[...] = (jnp.dot(h, w2_ref[...],
                              preferred_element_type=jnp.float32) + b2_ref[...])

    const = lambda i: (0, 0)
    return pl.pallas_call(
        body,
        grid=(BN // R,),
        in_specs=[
            pl.BlockSpec((R, 128), lambda i: (i, 0)),
            pl.BlockSpec((R, 128), lambda i: (i, 0)),
            pl.BlockSpec((R, 128), lambda i: (i, 0)),
            pl.BlockSpec((384, 128), const),
            pl.BlockSpec((1, 128), const),
            pl.BlockSpec((128, 128), const),
            pl.BlockSpec((1, 128), const),
        ],
        out_specs=pl.BlockSpec((R, 128), lambda i: (i, 0)),
        out_shape=jax.ShapeDtypeStruct((BN, 128), jnp.float32),
    )(*xs, w1, b1, w2, b2)


def kernel(xyzp, features, params):
    p = params
    nbatch, npts, _ = xyzp.shape
    BN = nbatch * npts
    nsamp = npts // 8
    BM = nbatch * nsamp
    f32 = jnp.float32

    xy = xyzp[..., :2]
    ptsT = jnp.concatenate(
        [jnp.swapaxes(xyzp, 1, 2), jnp.zeros((nbatch, 4, npts), f32)], axis=1)
    xcol = xyzp[..., 0].reshape(BN, 1)
    ycol = xyzp[..., 1].reshape(BN, 1)
    zcol = xyzp[..., 2].reshape(BN, 1)

    lw, lb = p['local_qkv_w'], p['local_qkv_b']
    cwf, cwp, cb = p['conv_qkv_wf'], p['conv_qkv_wp'], p['conv_qkv_b']
    gw, gb = p['global_qkv_w'], p['global_qkv_b']
    zf = jnp.zeros((128, 128), f32)
    zxy = jnp.zeros((2, 128), f32)
    zp = jnp.zeros((4, 128), f32)
    zb = jnp.zeros((128,), f32)
    band_f = jnp.concatenate([
        lw[:, 128:256], lw[:, 256:384], zf,
        cwf[:, 128:256], cwf[:, 256:384], zf,
        gw[:, 128:256], gw[:, 256:384],
        lw[:, 0:128], cwf[:, 0:128], gw[:, 0:128],
        zf, zf, zf,
        jnp.zeros((128, 256), f32)], axis=1)
    band_s = jnp.concatenate([
        jnp.concatenate([zxy, zxy, zxy,
                         cwp[:, 128:256], cwp[:, 256:384], p['conv_pe_w1'],
                         zxy, zxy,
                         zxy, cwp[:, 0:128], zxy,
                         zxy, p['conv_pe_w1'], zxy,
                         jnp.zeros((2, 256), f32)], axis=1),
        jnp.concatenate([zp, zp, p['local_pe_w1'],
                         zp, zp, zp,
                         zp, zp,
                         zp, zp, zp,
                         p['local_pe_w1'], zp, p['global_pe_w1'],
                         jnp.concatenate([jnp.zeros((4, 128), f32),
                                          p['global_pe_w1']], axis=1)],
                        axis=1),
        jnp.zeros((2, 2048), f32)], axis=0)
    bias = jnp.concatenate([
        lb[128:256], lb[256:384], zb,
        cb[128:256], cb[256:384], zb,
        gb[128:256], gb[256:384],
        lb[0:128], cb[0:128], gb[0:128],
        zb, zb, zb, jnp.zeros((256,), f32)])[None, :]
    xsmall = jnp.concatenate([xy.reshape(BN, 2), xyzp.reshape(BN, 4),
                              jnp.zeros((BN, 2), f32)], axis=1)
    table_l, table_c, table_g, qall, c1all, down_table = _qkv_call(
        features.reshape(BN, 128), xsmall, band_f, band_s, bias,
        (384, 384, 256, 384, 384, 256))

    r2 = (5.0 / 128.0) ** 2
    lb_idx = [_local_ball_idx_call(xcol, ycol, zcol, ptsT, r2, npts, b)
              for b in range(nbatch)]

    xsb = jnp.transpose(xyzp, (2, 0, 1)).reshape(16, npts)
    down_flat = jnp.transpose(_fps_call(xsb, nbatch, nsamp)[:, :nbatch],
                              (1, 0)).reshape(BM)
    dg = _sc_gather(down_table, down_flat)
    down16 = dg[:, :16]
    down_c1g = dg[:, 128:256]
    d16 = down16.reshape(nbatch, nsamp, 16)
    downT = jnp.swapaxes(d16, 1, 2)[:, :8, :]
    dxc = d16[..., 0].reshape(BM, 1)
    dyc = d16[..., 1].reshape(BM, 1)
    dzc = d16[..., 2].reshape(BM, 1)
    pair = jnp.transpose(
        _knn_call(xcol, ycol, zcol, downT, min(nsamp, 256), npts),
        (1, 0))
    inv = jnp.transpose(
        _knn_call(dxc, dyc, dzc, ptsT, 512, nsamp), (1, 0))

    gd = _sc_gather(table_g, pair.reshape(BM * _K))
    table_g2 = _maxpool_call(gd, 256, down_c1g)

    def attn(name, bidx, g, rb0):
        return _attn_call(
            qall, bidx, c1all, bidx, g,
            p[name + '_pe_b1'][None],
            p[name + '_pe_w2'], p[name + '_pe_b2'][None],
            p[name + '_fc_g'][None], p[name + '_fc_b'][None], rb0=rb0)

    nbb = npts // 256
    ge = _sc_gather(table_g2, inv.reshape(BN * _K))
    parts = {'local': [], 'conv': []}
    for b in range(nbatch):
        li_b, bi_b = lb_idx[b]
        ga_b = _sc_gather(table_l, li_b.reshape(npts * _K))
        gb_b = _sc_gather(table_c, bi_b.reshape(npts * _K))
        parts['local'].append(attn('local', 0, ga_b, b * nbb))
        parts['conv'].append(attn('conv', 1, gb_b, b * nbb))
    glob = attn('global', 2, ge, 0)
    local = jnp.concatenate(parts['local'], axis=0)
    conv = jnp.concatenate(parts['conv'], axis=0)

    out = _proj_call([local, conv, glob],
                     p['proj_w1'], p['proj_b1'][None],
                     p['proj_w2'], p['proj_b2'][None])
    return out.reshape(nbatch, npts, 128)
